# Initial kernel scaffold; baseline (speedup 1.0000x reference)
import jax
import jax.numpy as jnp
from jax.experimental import pallas as pl


def kernel(x, adj_t, edge_attr, W_q, b_q, bn_g_q, bn_b_q, W_n, b_n, bn_g_n, bn_b_n, W_t, b_t, bn_g_t, bn_b_t, graph_token):
    raise NotImplementedError("write your pallas kernel here")


# speedup vs baseline: 7.3990x; 7.3990x over previous
import functools

import jax
import jax.numpy as jnp
import numpy as np
from jax import lax
from jax.experimental import pallas as pl
from jax.experimental.pallas import tpu as pltpu
from jax.experimental.pallas import tpu_sc as plsc

_N = 10000
_E = 320000
_D = 128
_K = 8

_BN_C = float(1.0 / np.sqrt(1.0 + 1e-5))

_NC = 2
_NS = 16
_NW = _NC * _NS
_CH = 80
_NCHUNK = 128
_EPAD = _NW * _NCHUNK * _CH
_NP = 10240
_RPT = _NP // _NS
_ZR = 16


def _deg_body(dst_hbm, out_hbm, acc_sh, idx_d, ones_v, zbuf):
    c = lax.axis_index("c")
    s = lax.axis_index("s")
    for r in range(_CH):
        for j in range(_D // 16):
            ones_v[r, j * 16:(j + 1) * 16] = jnp.ones((16,), jnp.float32)
    for r in range(_ZR):
        for j in range(_D // 16):
            zbuf[r, j * 16:(j + 1) * 16] = jnp.zeros((16,), jnp.float32)

    def _zero(k, _):
        pltpu.sync_copy(zbuf, acc_sh.at[pl.ds(s * _RPT + k * _ZR, _ZR)])
        return 0

    lax.fori_loop(0, _RPT // _ZR, _zero, 0)

    base = (c * _NS + s) * _NCHUNK
    pltpu.sync_copy(dst_hbm.at[pl.ds(base, _NCHUNK)], idx_d)
    plsc.subcore_barrier()

    def _scat(j, _):
        pltpu.sync_copy(ones_v, acc_sh.at[idx_d.at[j]], add=True)
        return 0

    lax.fori_loop(0, _NCHUNK, _scat, 0)
    plsc.subcore_barrier()
    pltpu.sync_copy(acc_sh.at[pl.ds(s * _RPT, _RPT)],
                    out_hbm.at[c].at[pl.ds(s * _RPT, _RPT)])


def _prop_body(g_hbm, src_hbm, dst_hbm, out_hbm, acc_sh, idx_s, idx_d,
               rows_a, rows_b, zbuf, sem_a, sem_b):
    c = lax.axis_index("c")
    s = lax.axis_index("s")
    for r in range(_ZR):
        for j in range(_D // 16):
            zbuf[r, j * 16:(j + 1) * 16] = jnp.zeros((16,), jnp.float32)

    def _zero(k, _):
        pltpu.sync_copy(zbuf, acc_sh.at[pl.ds(s * _RPT + k * _ZR, _ZR)])
        return 0

    lax.fori_loop(0, _RPT // _ZR, _zero, 0)

    base = (c * _NS + s) * _NCHUNK
    plsc.subcore_barrier()

    half = _NCHUNK // 2
    for p in range(2):
        pltpu.sync_copy(src_hbm.at[pl.ds(base + p * half, half)], idx_s)
        pltpu.sync_copy(dst_hbm.at[pl.ds(base + p * half, half)], idx_d)
        pltpu.async_copy(g_hbm.at[idx_s.at[0]], rows_a, sem_a)
        pltpu.async_copy(g_hbm.at[idx_s.at[1]], rows_b, sem_b)

        def _pair(jj, _):
            j0 = jj * 2
            pltpu.make_async_copy(g_hbm.at[idx_s.at[j0]], rows_a, sem_a).wait()
            pltpu.sync_copy(rows_a, acc_sh.at[idx_d.at[j0]], add=True)
            pltpu.async_copy(g_hbm.at[idx_s.at[j0 + 2]], rows_a, sem_a)
            pltpu.make_async_copy(g_hbm.at[idx_s.at[j0 + 1]], rows_b,
                                  sem_b).wait()
            pltpu.sync_copy(rows_b, acc_sh.at[idx_d.at[j0 + 1]], add=True)
            pltpu.async_copy(g_hbm.at[idx_s.at[j0 + 3]], rows_b, sem_b)
            return 0

        lax.fori_loop(0, half // 2 - 1, _pair, 0)
        pltpu.make_async_copy(g_hbm.at[idx_s.at[half - 2]], rows_a, sem_a).wait()
        pltpu.sync_copy(rows_a, acc_sh.at[idx_d.at[half - 2]], add=True)
        pltpu.make_async_copy(g_hbm.at[idx_s.at[half - 1]], rows_b, sem_b).wait()
        pltpu.sync_copy(rows_b, acc_sh.at[idx_d.at[half - 1]], add=True)

    plsc.subcore_barrier()
    pltpu.sync_copy(acc_sh.at[pl.ds(s * _RPT, _RPT)],
                    out_hbm.at[c].at[pl.ds(s * _RPT, _RPT)])


@functools.lru_cache(maxsize=None)
def _make_sc_kernels():
    sc_mesh = plsc.VectorSubcoreMesh(core_axis_name="c", subcore_axis_name="s",
                                     num_cores=_NC, num_subcores=_NS)
    deg = pl.kernel(
        _deg_body,
        jax.ShapeDtypeStruct((_NC, _NP, _D), jnp.float32),
        mesh=sc_mesh,
        scratch_types=[
            pltpu.VMEM_SHARED((_NP, _D), jnp.float32),
            pltpu.VMEM((_NCHUNK, _CH), jnp.int32),
            pltpu.VMEM((_CH, _D), jnp.float32),
            pltpu.VMEM((_ZR, _D), jnp.float32),
        ],
    )
    prop = pl.kernel(
        _prop_body,
        jax.ShapeDtypeStruct((_NC, _NP, _D), jnp.float32),
        mesh=sc_mesh,
        scratch_types=[
            pltpu.VMEM_SHARED((_NP, _D), jnp.float32),
            pltpu.VMEM((_NCHUNK // 2, _CH), jnp.int32),
            pltpu.VMEM((_NCHUNK // 2, _CH), jnp.int32),
            pltpu.VMEM((_CH, _D), jnp.float32),
            pltpu.VMEM((_CH, _D), jnp.float32),
            pltpu.VMEM((_ZR, _D), jnp.float32),
            pltpu.SemaphoreType.DMA,
            pltpu.SemaphoreType.DMA,
        ],
    )
    return deg, prop


_BLK = 1000
_GRID = _N // _BLK


def _prelude_tc(degp_ref, x_ref, w_ref, dinv_ref, g_ref):
    deg = 1.0 + degp_ref[0, :, 0:1] + degp_ref[1, :, 0:1]
    dinv = lax.rsqrt(deg)
    dinv_ref[...] = dinv
    g_ref[...] = dinv * jnp.dot(x_ref[...], w_ref[...],
                                preferred_element_type=jnp.float32)


def _prelude(degp, x, w):
    return pl.pallas_call(
        _prelude_tc,
        grid=(_GRID,),
        in_specs=[
            pl.BlockSpec((_NC, _BLK, _D), lambda i: (0, i, 0)),
            pl.BlockSpec((_BLK, _D), lambda i: (i, 0)),
            pl.BlockSpec((_D, _D), lambda i: (0, 0)),
        ],
        out_specs=[
            pl.BlockSpec((_BLK, 1), lambda i: (i, 0)),
            pl.BlockSpec((_BLK, _D), lambda i: (i, 0)),
        ],
        out_shape=[
            jax.ShapeDtypeStruct((_N, 1), jnp.float32),
            jax.ShapeDtypeStruct((_N, _D), jnp.float32),
        ],
    )(degp, x, w)


def _step_tc(with_bn, parts_ref, g_ref, dinv_ref, b_ref, gam_ref, bet_ref,
             w_ref, out_ref):
    dinv = dinv_ref[...]
    o = dinv * (parts_ref[0] + parts_ref[1] + g_ref[...]) + b_ref[...]
    if with_bn:
        o = jnp.maximum(o * (gam_ref[...] * _BN_C) + bet_ref[...], 0.0)
    out_ref[...] = dinv * jnp.dot(o, w_ref[...],
                                  preferred_element_type=jnp.float32)


def _step(parts, g, dinv, b, gam, bet, w, with_bn):
    return pl.pallas_call(
        functools.partial(_step_tc, with_bn),
        grid=(_GRID,),
        in_specs=[
            pl.BlockSpec((_NC, _BLK, _D), lambda i: (0, i, 0)),
            pl.BlockSpec((_BLK, _D), lambda i: (i, 0)),
            pl.BlockSpec((_BLK, 1), lambda i: (i, 0)),
            pl.BlockSpec((1, _D), lambda i: (0, 0)),
            pl.BlockSpec((1, _D), lambda i: (0, 0)),
            pl.BlockSpec((1, _D), lambda i: (0, 0)),
            pl.BlockSpec((_D, _D), lambda i: (0, 0)),
        ],
        out_specs=pl.BlockSpec((_BLK, _D), lambda i: (i, 0)),
        out_shape=jax.ShapeDtypeStruct((_N, _D), jnp.float32),
    )(parts, g, dinv, b.reshape(1, _D), gam.reshape(1, _D),
      bet.reshape(1, _D), w)


def _stage3_tc(parts_ref, g_ref, dinv_ref, bprev_ref, w0_ref, b0_ref,
               gam_ref, bet_ref, w1_ref, b1_ref, tok_ref, out_ref,
               s1_acc, cs_acc):
    i = pl.program_id(0)

    @pl.when(i == 0)
    def _():
        s1_acc[...] = jnp.zeros_like(s1_acc)
        cs_acc[...] = jnp.zeros_like(cs_acc)

    h4 = dinv_ref[...] * (parts_ref[0] + parts_ref[1] + g_ref[...]) \
        + bprev_ref[...]
    s1_acc[...] += jnp.sum(h4, axis=0, keepdims=True)
    t = jnp.dot(h4, w0_ref[...], preferred_element_type=jnp.float32) \
        + b0_ref[...]
    v = jnp.maximum(t * (gam_ref[...] * _BN_C) + bet_ref[...], 0.0)
    cs_acc[...] += jnp.sum(v, axis=0, keepdims=True)

    @pl.when(i == _GRID - 1)
    def _():
        n1 = float(_N + 1)
        isq = float(1.0 / np.sqrt(n1))
        inv = float(1.0 / n1)
        u_tok = jnp.dot(s1_acc[...] * isq, w0_ref[...],
                        preferred_element_type=jnp.float32) \
            + jnp.dot(tok_ref[...], w0_ref[...],
                      preferred_element_type=jnp.float32) * inv \
            + b0_ref[...]
        v_tok = jnp.maximum(u_tok * (gam_ref[...] * _BN_C) + bet_ref[...], 0.0)
        out_ref[...] = jnp.dot(cs_acc[...] * isq, w1_ref[...],
                               preferred_element_type=jnp.float32) \
            + jnp.dot(v_tok, w1_ref[...],
                      preferred_element_type=jnp.float32) * inv \
            + b1_ref[...]


def _stage3(parts, g, dinv, bprev, w0, b0, gam, bet, w1, b1, tok):
    full = lambda i: (0, 0)
    return pl.pallas_call(
        _stage3_tc,
        grid=(_GRID,),
        in_specs=[
            pl.BlockSpec((_NC, _BLK, _D), lambda i: (0, i, 0)),
            pl.BlockSpec((_BLK, _D), lambda i: (i, 0)),
            pl.BlockSpec((_BLK, 1), lambda i: (i, 0)),
            pl.BlockSpec((1, _D), full),
            pl.BlockSpec((_D, _D), full),
            pl.BlockSpec((1, _D), full),
            pl.BlockSpec((1, _D), full),
            pl.BlockSpec((1, _D), full),
            pl.BlockSpec((_D, _D), full),
            pl.BlockSpec((1, _D), full),
            pl.BlockSpec((_K, _D), full),
        ],
        out_specs=pl.BlockSpec((_K, _D), full),
        out_shape=jax.ShapeDtypeStruct((_K, _D), jnp.float32),
        scratch_shapes=[
            pltpu.VMEM((1, _D), jnp.float32),
            pltpu.VMEM((1, _D), jnp.float32),
        ],
    )(parts, g, dinv, bprev.reshape(1, _D), w0, b0.reshape(1, _D),
      gam.reshape(1, _D), bet.reshape(1, _D), w1, b1.reshape(1, _D), tok)


def kernel(x, adj_t, edge_attr, W_q, b_q, bn_g_q, bn_b_q, W_n, b_n, bn_g_n,
           bn_b_n, W_t, b_t, bn_g_t, bn_b_t, graph_token):
    del edge_attr
    _deg_sc, _prop_sc = _make_sc_kernels()

    npad = _EPAD - _E
    src2d = jnp.concatenate(
        [adj_t[0], jnp.zeros((npad,), jnp.int32)]).reshape(_EPAD // _CH, _CH)
    dst2d = jnp.concatenate(
        [adj_t[1], jnp.full((npad,), _NP - 1, jnp.int32)]).reshape(
            _EPAD // _CH, _CH)

    degp = _deg_sc(dst2d)
    dinv, g = _prelude(degp, x, W_q[0])

    parts = _prop_sc(g, src2d, dst2d)
    g = _step(parts, g, dinv, b_q[0], bn_g_q[0], bn_b_q[0], W_q[1], True)
    parts = _prop_sc(g, src2d, dst2d)
    g = _step(parts, g, dinv, b_q[1], b_q[1], b_q[1], W_n[0], False)
    parts = _prop_sc(g, src2d, dst2d)
    g = _step(parts, g, dinv, b_n[0], bn_g_n[0], bn_b_n[0], W_n[1], True)
    parts = _prop_sc(g, src2d, dst2d)
    g = _step(parts, g, dinv, b_n[1], bn_g_n[1], bn_b_n[1], W_n[2], True)
    parts = _prop_sc(g, src2d, dst2d)

    return _stage3(parts, g, dinv, b_n[2], W_t[0], b_t[0], bn_g_t[0],
                   bn_b_t[0], W_t[1], b_t[1], graph_token)


# speedup vs baseline: 15.5548x; 2.1023x over previous
import functools

import jax
import jax.numpy as jnp
import numpy as np
from jax import lax
from jax.experimental import pallas as pl
from jax.experimental.pallas import tpu as pltpu
from jax.experimental.pallas import tpu_sc as plsc

_N = 10000
_E = 320000
_D = 128
_H = _D // 2
_K = 8

_BN_C = float(1.0 / np.sqrt(1.0 + 1e-5))

_NC = 2
_NS = 16
_NW = _NC * _NS
_CH = 80
_NCH = 256
_EPAD = _NS * _NCH * _CH
_NP = 10112
_RPT = _NP // _NS
_ZR = 8
_NBUF = 4
_PCH = 16


_DNCH = _EPAD // _NW // _CH


def _deg_body(dst_hbm, out_hbm, acc_sh, idx_d, ones_v, zbuf):
    c = lax.axis_index("c")
    s = lax.axis_index("s")
    for r in range(_CH):
        for j in range(_D // 16):
            ones_v[r, j * 16:(j + 1) * 16] = jnp.ones((16,), jnp.float32)
    for r in range(_ZR):
        for j in range(_D // 16):
            zbuf[r, j * 16:(j + 1) * 16] = jnp.zeros((16,), jnp.float32)

    def _zero(k, _):
        pltpu.sync_copy(zbuf, acc_sh.at[pl.ds(s * _RPT + k * _ZR, _ZR)])
        return 0

    lax.fori_loop(0, _RPT // _ZR, _zero, 0)

    base = (c * _NS + s) * _DNCH
    pltpu.sync_copy(dst_hbm.at[pl.ds(base, _DNCH)], idx_d)
    plsc.subcore_barrier()

    def _scat(j, _):
        pltpu.sync_copy(ones_v, acc_sh.at[idx_d.at[j]], add=True)
        return 0

    lax.fori_loop(0, _DNCH, _scat, 0)
    plsc.subcore_barrier()
    pltpu.sync_copy(acc_sh.at[pl.ds(s * _RPT, _RPT)],
                    out_hbm.at[c].at[pl.ds(s * _RPT, _RPT)])


def _prop_body(g_hbm, src_hbm, dst_hbm, out_hbm, g_sh, acc_sh, idx_s, idx_d,
               rows0, rows1, rows2, rows3, zbuf,
               gsem0, gsem1, gsem2, gsem3, ssem0, ssem1, ssem2, ssem3):
    c = lax.axis_index("c")
    s = lax.axis_index("s")
    rows = [rows0, rows1, rows2, rows3]
    gsem = [gsem0, gsem1, gsem2, gsem3]
    ssem = [ssem0, ssem1, ssem2, ssem3]
    for r in range(_ZR):
        for j in range(_H // 16):
            zbuf[r, j * 16:(j + 1) * 16] = jnp.zeros((16,), jnp.float32)
    arow = s * _RPT

    def _zero(k, _):
        pltpu.sync_copy(zbuf, acc_sh.at[pl.ds(arow + k * _ZR, _ZR)])
        return 0

    lax.fori_loop(0, _RPT // _ZR, _zero, 0)
    pltpu.sync_copy(g_hbm.at[c].at[pl.ds(arow, _RPT)], g_sh.at[pl.ds(arow, _RPT)])
    base = s * _NCH
    plsc.subcore_barrier()

    def _phase(p, _):
        pltpu.sync_copy(src_hbm.at[pl.ds(base + p * _PCH, _PCH)], idx_s)
        pltpu.sync_copy(dst_hbm.at[pl.ds(base + p * _PCH, _PCH)], idx_d)
        for b in range(_NBUF):
            pltpu.async_copy(g_sh.at[idx_s.at[b]], rows[b], gsem[b])

        def _quad(qq, _):
            l0 = qq * _NBUF
            for b in range(_NBUF):
                pltpu.make_async_copy(g_sh.at[idx_s.at[l0 + b]], rows[b],
                                      gsem[b]).wait()
                pltpu.async_copy(rows[b], acc_sh.at[idx_d.at[l0 + b]],
                                 ssem[b], add=True)
            for b in range(_NBUF):
                pltpu.make_async_copy(rows[b], acc_sh.at[idx_d.at[l0 + b]],
                                      ssem[b]).wait()
                pltpu.async_copy(g_sh.at[idx_s.at[l0 + _NBUF + b]], rows[b],
                                 gsem[b])
            return 0

        lax.fori_loop(0, _PCH // _NBUF - 1, _quad, 0)
        lt = _PCH - _NBUF
        for b in range(_NBUF):
            pltpu.make_async_copy(g_sh.at[idx_s.at[lt + b]], rows[b],
                                  gsem[b]).wait()
            pltpu.async_copy(rows[b], acc_sh.at[idx_d.at[lt + b]],
                             ssem[b], add=True)
        for b in range(_NBUF):
            pltpu.make_async_copy(rows[b], acc_sh.at[idx_d.at[lt + b]],
                                  ssem[b]).wait()
        return 0

    lax.fori_loop(0, _NCH // _PCH, _phase, 0)

    plsc.subcore_barrier()
    pltpu.sync_copy(acc_sh.at[pl.ds(arow, _RPT)],
                    out_hbm.at[c].at[pl.ds(arow, _RPT)])


@functools.lru_cache(maxsize=None)
def _make_sc_kernels():
    sc_mesh = plsc.VectorSubcoreMesh(core_axis_name="c", subcore_axis_name="s",
                                     num_cores=_NC, num_subcores=_NS)
    deg = pl.kernel(
        _deg_body,
        jax.ShapeDtypeStruct((_NC, _NP, _D), jnp.float32),
        mesh=sc_mesh,
        scratch_types=[
            pltpu.VMEM_SHARED((_NP, _D), jnp.float32),
            pltpu.VMEM((_DNCH, _CH), jnp.int32),
            pltpu.VMEM((_CH, _D), jnp.float32),
            pltpu.VMEM((_ZR, _D), jnp.float32),
        ],
    )
    prop = pl.kernel(
        _prop_body,
        jax.ShapeDtypeStruct((_NC, _NP, _H), jnp.float32),
        mesh=sc_mesh,
        compiler_params=pltpu.CompilerParams(use_tc_tiling_on_sc=False),
        scratch_types=[
            pltpu.VMEM_SHARED((_NP, _H), jnp.float32),
            pltpu.VMEM_SHARED((_NP, _H), jnp.float32),
            pltpu.VMEM((_PCH, _CH), jnp.int32),
            pltpu.VMEM((_PCH, _CH), jnp.int32),
            pltpu.VMEM((_CH, _H), jnp.float32),
            pltpu.VMEM((_CH, _H), jnp.float32),
            pltpu.VMEM((_CH, _H), jnp.float32),
            pltpu.VMEM((_CH, _H), jnp.float32),
            pltpu.VMEM((_ZR, _H), jnp.float32),
        ] + [pltpu.SemaphoreType.DMA] * 8,
    )
    return deg, prop


_BLK = 1000
_GRID = _N // _BLK


def _prelude_tc(degp_ref, x_ref, w_ref, dinv_ref, g_ref):
    deg = 1.0 + degp_ref[0, :, 0:1] + degp_ref[1, :, 0:1]
    dinv = lax.rsqrt(deg)
    dinv_ref[...] = dinv
    h = jnp.dot(x_ref[...], w_ref[...], preferred_element_type=jnp.float32)
    g_ref[0] = dinv * h[:, :_H]
    g_ref[1] = dinv * h[:, _H:]


def _prelude(degp, x, w):
    return pl.pallas_call(
        _prelude_tc,
        grid=(_GRID,),
        in_specs=[
            pl.BlockSpec((_NC, _BLK, _D), lambda i: (0, i, 0)),
            pl.BlockSpec((_BLK, _D), lambda i: (i, 0)),
            pl.BlockSpec((_D, _D), lambda i: (0, 0)),
        ],
        out_specs=[
            pl.BlockSpec((_BLK, 1), lambda i: (i, 0)),
            pl.BlockSpec((_NC, _BLK, _H), lambda i: (0, i, 0)),
        ],
        out_shape=[
            jax.ShapeDtypeStruct((_N, 1), jnp.float32),
            jax.ShapeDtypeStruct((_NC, _NP, _H), jnp.float32),
        ],
    )(degp, x, w)


def _step_tc(with_bn, parts_ref, g_ref, dinv_ref, b_ref, gam_ref, bet_ref,
             w_ref, out_ref):
    dinv = dinv_ref[...]
    o0 = dinv * (parts_ref[0] + g_ref[0]) + b_ref[:, :_H]
    o1 = dinv * (parts_ref[1] + g_ref[1]) + b_ref[:, _H:]
    if with_bn:
        o0 = jnp.maximum(o0 * (gam_ref[:, :_H] * _BN_C) + bet_ref[:, :_H], 0.0)
        o1 = jnp.maximum(o1 * (gam_ref[:, _H:] * _BN_C) + bet_ref[:, _H:], 0.0)
    h = jnp.dot(o0, w_ref[:_H, :], preferred_element_type=jnp.float32) \
        + jnp.dot(o1, w_ref[_H:, :], preferred_element_type=jnp.float32)
    out_ref[0] = dinv * h[:, :_H]
    out_ref[1] = dinv * h[:, _H:]


def _step(parts, g, dinv, b, gam, bet, w, with_bn):
    return pl.pallas_call(
        functools.partial(_step_tc, with_bn),
        grid=(_GRID,),
        in_specs=[
            pl.BlockSpec((_NC, _BLK, _H), lambda i: (0, i, 0)),
            pl.BlockSpec((_NC, _BLK, _H), lambda i: (0, i, 0)),
            pl.BlockSpec((_BLK, 1), lambda i: (i, 0)),
            pl.BlockSpec((1, _D), lambda i: (0, 0)),
            pl.BlockSpec((1, _D), lambda i: (0, 0)),
            pl.BlockSpec((1, _D), lambda i: (0, 0)),
            pl.BlockSpec((_D, _D), lambda i: (0, 0)),
        ],
        out_specs=pl.BlockSpec((_NC, _BLK, _H), lambda i: (0, i, 0)),
        out_shape=jax.ShapeDtypeStruct((_NC, _NP, _H), jnp.float32),
    )(parts, g, dinv, b.reshape(1, _D), gam.reshape(1, _D),
      bet.reshape(1, _D), w)


def _stage3_tc(parts_ref, g_ref, dinv_ref, bprev_ref, w0_ref, b0_ref,
               gam_ref, bet_ref, w1_ref, b1_ref, tok_ref, out_ref,
               s1_acc, cs_acc):
    i = pl.program_id(0)

    @pl.when(i == 0)
    def _():
        s1_acc[...] = jnp.zeros_like(s1_acc)
        cs_acc[...] = jnp.zeros_like(cs_acc)

    dinv = dinv_ref[...]
    h40 = dinv * (parts_ref[0] + g_ref[0]) + bprev_ref[:, :_H]
    h41 = dinv * (parts_ref[1] + g_ref[1]) + bprev_ref[:, _H:]
    s1_acc[:, :_H] += jnp.sum(h40, axis=0, keepdims=True)
    s1_acc[:, _H:] += jnp.sum(h41, axis=0, keepdims=True)
    t = jnp.dot(h40, w0_ref[:_H, :], preferred_element_type=jnp.float32) \
        + jnp.dot(h41, w0_ref[_H:, :], preferred_element_type=jnp.float32) \
        + b0_ref[...]
    v = jnp.maximum(t * (gam_ref[...] * _BN_C) + bet_ref[...], 0.0)
    cs_acc[...] += jnp.sum(v, axis=0, keepdims=True)

    @pl.when(i == _GRID - 1)
    def _():
        n1 = float(_N + 1)
        isq = float(1.0 / np.sqrt(n1))
        inv = float(1.0 / n1)
        u_tok = jnp.dot(s1_acc[...] * isq, w0_ref[...],
                        preferred_element_type=jnp.float32) \
            + jnp.dot(tok_ref[...], w0_ref[...],
                      preferred_element_type=jnp.float32) * inv \
            + b0_ref[...]
        v_tok = jnp.maximum(u_tok * (gam_ref[...] * _BN_C) + bet_ref[...], 0.0)
        out_ref[...] = jnp.dot(cs_acc[...] * isq, w1_ref[...],
                               preferred_element_type=jnp.float32) \
            + jnp.dot(v_tok, w1_ref[...],
                      preferred_element_type=jnp.float32) * inv \
            + b1_ref[...]


def _stage3(parts, g, dinv, bprev, w0, b0, gam, bet, w1, b1, tok):
    full = lambda i: (0, 0)
    return pl.pallas_call(
        _stage3_tc,
        grid=(_GRID,),
        in_specs=[
            pl.BlockSpec((_NC, _BLK, _H), lambda i: (0, i, 0)),
            pl.BlockSpec((_NC, _BLK, _H), lambda i: (0, i, 0)),
            pl.BlockSpec((_BLK, 1), lambda i: (i, 0)),
            pl.BlockSpec((1, _D), full),
            pl.BlockSpec((_D, _D), full),
            pl.BlockSpec((1, _D), full),
            pl.BlockSpec((1, _D), full),
            pl.BlockSpec((1, _D), full),
            pl.BlockSpec((_D, _D), full),
            pl.BlockSpec((1, _D), full),
            pl.BlockSpec((_K, _D), full),
        ],
        out_specs=pl.BlockSpec((_K, _D), full),
        out_shape=jax.ShapeDtypeStruct((_K, _D), jnp.float32),
        scratch_shapes=[
            pltpu.VMEM((1, _D), jnp.float32),
            pltpu.VMEM((1, _D), jnp.float32),
        ],
    )(parts, g, dinv, bprev.reshape(1, _D), w0, b0.reshape(1, _D),
      gam.reshape(1, _D), bet.reshape(1, _D), w1, b1.reshape(1, _D), tok)


def kernel(x, adj_t, edge_attr, W_q, b_q, bn_g_q, bn_b_q, W_n, b_n, bn_g_n,
           bn_b_n, W_t, b_t, bn_g_t, bn_b_t, graph_token):
    del edge_attr
    _deg_sc, _prop_sc = _make_sc_kernels()

    npad = _EPAD - _E
    src2d = jnp.concatenate(
        [adj_t[0], jnp.zeros((npad,), jnp.int32)]).reshape(_EPAD // _CH, _CH)
    dst2d = jnp.concatenate(
        [adj_t[1], jnp.full((npad,), _NP - 1, jnp.int32)]).reshape(
            _EPAD // _CH, _CH)

    degp = _deg_sc(dst2d)
    dinv, g = _prelude(degp, x, W_q[0])

    parts = _prop_sc(g, src2d, dst2d)
    g = _step(parts, g, dinv, b_q[0], bn_g_q[0], bn_b_q[0], W_q[1], True)
    parts = _prop_sc(g, src2d, dst2d)
    g = _step(parts, g, dinv, b_q[1], b_q[1], b_q[1], W_n[0], False)
    parts = _prop_sc(g, src2d, dst2d)
    g = _step(parts, g, dinv, b_n[0], bn_g_n[0], bn_b_n[0], W_n[1], True)
    parts = _prop_sc(g, src2d, dst2d)
    g = _step(parts, g, dinv, b_n[1], bn_g_n[1], bn_b_n[1], W_n[2], True)
    parts = _prop_sc(g, src2d, dst2d)

    return _stage3(parts, g, dinv, b_n[2], W_t[0], b_t[0], bn_g_t[0],
                   bn_b_t[0], W_t[1], b_t[1], graph_token)


# speedup vs baseline: 16.4802x; 1.0595x over previous
import functools

import jax
import jax.numpy as jnp
import numpy as np
from jax import lax
from jax.experimental import pallas as pl
from jax.experimental.pallas import tpu as pltpu
from jax.experimental.pallas import tpu_sc as plsc

_N = 10000
_E = 320000
_D = 128
_H = _D // 2
_K = 8

_BN_C = float(1.0 / np.sqrt(1.0 + 1e-5))

_NC = 2
_NS = 16
_NW = _NC * _NS
_CH = 128
_NCH = 160
_EPAD = _NS * _NCH * _CH
_NP = 10112
_RPT = _NP // _NS
_ZR = 8
_ZB = 40
_NBUF = 4
_PCH = 16


_DNCH = _EPAD // _NW // _CH


def _deg_body(dst_hbm, out_hbm, acc_sh, idx_d, ones_v, zbuf):
    c = lax.axis_index("c")
    s = lax.axis_index("s")
    for r in range(_CH):
        for j in range(_D // 16):
            ones_v[r, j * 16:(j + 1) * 16] = jnp.ones((16,), jnp.float32)
    for r in range(_ZR):
        for j in range(_D // 16):
            zbuf[r, j * 16:(j + 1) * 16] = jnp.zeros((16,), jnp.float32)

    def _zero(k, _):
        pltpu.sync_copy(zbuf, acc_sh.at[pl.ds(s * _RPT + k * _ZR, _ZR)])
        return 0

    lax.fori_loop(0, _RPT // _ZR, _zero, 0)

    base = (c * _NS + s) * _DNCH
    pltpu.sync_copy(dst_hbm.at[pl.ds(base, _DNCH)], idx_d)
    plsc.subcore_barrier()

    def _scat(j, _):
        pltpu.sync_copy(ones_v, acc_sh.at[idx_d.at[j]], add=True)
        return 0

    lax.fori_loop(0, _DNCH, _scat, 0)
    plsc.subcore_barrier()
    pltpu.sync_copy(acc_sh.at[pl.ds(s * _RPT, _RPT)],
                    out_hbm.at[c].at[pl.ds(s * _RPT, _RPT)])


def _prop_body(g_hbm, src_hbm, dst_hbm, out_hbm, g_sh, acc_sh, idx_s, idx_d,
               rows0, rows1, rows2, rows3, zbuf,
               gsem0, gsem1, gsem2, gsem3, ssem0, ssem1, ssem2, ssem3):
    c = lax.axis_index("c")
    s = lax.axis_index("s")
    rows = [rows0, rows1, rows2, rows3]
    gsem = [gsem0, gsem1, gsem2, gsem3]
    ssem = [ssem0, ssem1, ssem2, ssem3]
    for r in range(_ZB):
        for j in range(_H // 16):
            zbuf[r, j * 16:(j + 1) * 16] = jnp.zeros((16,), jnp.float32)
    arow = s * _RPT
    for k in range(_RPT // _ZB):
        pltpu.sync_copy(zbuf, acc_sh.at[pl.ds(arow + k * _ZB, _ZB)])
    if _RPT % _ZB:
        pltpu.sync_copy(zbuf.at[pl.ds(0, _RPT % _ZB)],
                        acc_sh.at[pl.ds(arow + (_RPT // _ZB) * _ZB,
                                        _RPT % _ZB)])
    pltpu.sync_copy(g_hbm.at[c].at[pl.ds(arow, _RPT)], g_sh.at[pl.ds(arow, _RPT)])
    base = s * _NCH
    plsc.subcore_barrier()

    def _phase(p, _):
        pltpu.sync_copy(src_hbm.at[pl.ds(base + p * _PCH, _PCH)], idx_s)
        pltpu.sync_copy(dst_hbm.at[pl.ds(base + p * _PCH, _PCH)], idx_d)
        for b in range(_NBUF):
            pltpu.async_copy(g_sh.at[idx_s.at[b]], rows[b], gsem[b])

        def _quad(qq, _):
            l0 = qq * _NBUF
            for b in range(_NBUF):
                pltpu.make_async_copy(g_sh.at[idx_s.at[l0 + b]], rows[b],
                                      gsem[b]).wait()
                pltpu.async_copy(rows[b], acc_sh.at[idx_d.at[l0 + b]],
                                 ssem[b], add=True)
            for b in range(_NBUF):
                pltpu.make_async_copy(rows[b], acc_sh.at[idx_d.at[l0 + b]],
                                      ssem[b]).wait()
                pltpu.async_copy(g_sh.at[idx_s.at[l0 + _NBUF + b]], rows[b],
                                 gsem[b])
            return 0

        lax.fori_loop(0, _PCH // _NBUF - 1, _quad, 0)
        lt = _PCH - _NBUF
        for b in range(_NBUF):
            pltpu.make_async_copy(g_sh.at[idx_s.at[lt + b]], rows[b],
                                  gsem[b]).wait()
            pltpu.async_copy(rows[b], acc_sh.at[idx_d.at[lt + b]],
                             ssem[b], add=True)
        for b in range(_NBUF):
            pltpu.make_async_copy(rows[b], acc_sh.at[idx_d.at[lt + b]],
                                  ssem[b]).wait()
        return 0

    lax.fori_loop(0, _NCH // _PCH, _phase, 0)

    plsc.subcore_barrier()
    pltpu.sync_copy(acc_sh.at[pl.ds(arow, _RPT)],
                    out_hbm.at[c].at[pl.ds(arow, _RPT)])


@functools.lru_cache(maxsize=None)
def _make_sc_kernels():
    sc_mesh = plsc.VectorSubcoreMesh(core_axis_name="c", subcore_axis_name="s",
                                     num_cores=_NC, num_subcores=_NS)
    deg = pl.kernel(
        _deg_body,
        jax.ShapeDtypeStruct((_NC, _NP, _D), jnp.float32),
        mesh=sc_mesh,
        scratch_types=[
            pltpu.VMEM_SHARED((_NP, _D), jnp.float32),
            pltpu.VMEM((_DNCH, _CH), jnp.int32),
            pltpu.VMEM((_CH, _D), jnp.float32),
            pltpu.VMEM((_ZR, _D), jnp.float32),
        ],
    )
    prop = pl.kernel(
        _prop_body,
        jax.ShapeDtypeStruct((_NC, _NP, _H), jnp.float32),
        mesh=sc_mesh,
        compiler_params=pltpu.CompilerParams(use_tc_tiling_on_sc=False),
        scratch_types=[
            pltpu.VMEM_SHARED((_NP, _H), jnp.float32),
            pltpu.VMEM_SHARED((_NP, _H), jnp.float32),
            pltpu.VMEM((_PCH, _CH), jnp.int32),
            pltpu.VMEM((_PCH, _CH), jnp.int32),
            pltpu.VMEM((_CH, _H), jnp.float32),
            pltpu.VMEM((_CH, _H), jnp.float32),
            pltpu.VMEM((_CH, _H), jnp.float32),
            pltpu.VMEM((_CH, _H), jnp.float32),
            pltpu.VMEM((_ZB, _H), jnp.float32),
        ] + [pltpu.SemaphoreType.DMA] * 8,
    )
    return deg, prop


_BLK = 1000
_GRID = _N // _BLK


def _prelude_tc(degp_ref, x_ref, w_ref, dinv_ref, g_ref):
    deg = 1.0 + degp_ref[0, :, 0:1] + degp_ref[1, :, 0:1]
    dinv = lax.rsqrt(deg)
    dinv_ref[...] = dinv
    h = jnp.dot(x_ref[...], w_ref[...], preferred_element_type=jnp.float32)
    g_ref[0] = dinv * h[:, :_H]
    g_ref[1] = dinv * h[:, _H:]


def _prelude(degp, x, w):
    return pl.pallas_call(
        _prelude_tc,
        grid=(_GRID,),
        in_specs=[
            pl.BlockSpec((_NC, _BLK, _D), lambda i: (0, i, 0)),
            pl.BlockSpec((_BLK, _D), lambda i: (i, 0)),
            pl.BlockSpec((_D, _D), lambda i: (0, 0)),
        ],
        out_specs=[
            pl.BlockSpec((_BLK, 1), lambda i: (i, 0)),
            pl.BlockSpec((_NC, _BLK, _H), lambda i: (0, i, 0)),
        ],
        out_shape=[
            jax.ShapeDtypeStruct((_N, 1), jnp.float32),
            jax.ShapeDtypeStruct((_NC, _NP, _H), jnp.float32),
        ],
    )(degp, x, w)


def _step_tc(with_bn, parts_ref, g_ref, dinv_ref, b_ref, gam_ref, bet_ref,
             w_ref, out_ref):
    dinv = dinv_ref[...]
    o0 = dinv * (parts_ref[0] + g_ref[0]) + b_ref[:, :_H]
    o1 = dinv * (parts_ref[1] + g_ref[1]) + b_ref[:, _H:]
    if with_bn:
        o0 = jnp.maximum(o0 * (gam_ref[:, :_H] * _BN_C) + bet_ref[:, :_H], 0.0)
        o1 = jnp.maximum(o1 * (gam_ref[:, _H:] * _BN_C) + bet_ref[:, _H:], 0.0)
    h = jnp.dot(o0, w_ref[:_H, :], preferred_element_type=jnp.float32) \
        + jnp.dot(o1, w_ref[_H:, :], preferred_element_type=jnp.float32)
    out_ref[0] = dinv * h[:, :_H]
    out_ref[1] = dinv * h[:, _H:]


def _step(parts, g, dinv, b, gam, bet, w, with_bn):
    return pl.pallas_call(
        functools.partial(_step_tc, with_bn),
        grid=(_GRID,),
        in_specs=[
            pl.BlockSpec((_NC, _BLK, _H), lambda i: (0, i, 0)),
            pl.BlockSpec((_NC, _BLK, _H), lambda i: (0, i, 0)),
            pl.BlockSpec((_BLK, 1), lambda i: (i, 0)),
            pl.BlockSpec((1, _D), lambda i: (0, 0)),
            pl.BlockSpec((1, _D), lambda i: (0, 0)),
            pl.BlockSpec((1, _D), lambda i: (0, 0)),
            pl.BlockSpec((_D, _D), lambda i: (0, 0)),
        ],
        out_specs=pl.BlockSpec((_NC, _BLK, _H), lambda i: (0, i, 0)),
        out_shape=jax.ShapeDtypeStruct((_NC, _NP, _H), jnp.float32),
    )(parts, g, dinv, b.reshape(1, _D), gam.reshape(1, _D),
      bet.reshape(1, _D), w)


def _stage3_tc(parts_ref, g_ref, dinv_ref, bprev_ref, w0_ref, b0_ref,
               gam_ref, bet_ref, w1_ref, b1_ref, tok_ref, out_ref,
               s1_acc, cs_acc):
    i = pl.program_id(0)

    @pl.when(i == 0)
    def _():
        s1_acc[...] = jnp.zeros_like(s1_acc)
        cs_acc[...] = jnp.zeros_like(cs_acc)

    dinv = dinv_ref[...]
    h40 = dinv * (parts_ref[0] + g_ref[0]) + bprev_ref[:, :_H]
    h41 = dinv * (parts_ref[1] + g_ref[1]) + bprev_ref[:, _H:]
    s1_acc[:, :_H] += jnp.sum(h40, axis=0, keepdims=True)
    s1_acc[:, _H:] += jnp.sum(h41, axis=0, keepdims=True)
    t = jnp.dot(h40, w0_ref[:_H, :], preferred_element_type=jnp.float32) \
        + jnp.dot(h41, w0_ref[_H:, :], preferred_element_type=jnp.float32) \
        + b0_ref[...]
    v = jnp.maximum(t * (gam_ref[...] * _BN_C) + bet_ref[...], 0.0)
    cs_acc[...] += jnp.sum(v, axis=0, keepdims=True)

    @pl.when(i == _GRID - 1)
    def _():
        n1 = float(_N + 1)
        isq = float(1.0 / np.sqrt(n1))
        inv = float(1.0 / n1)
        u_tok = jnp.dot(s1_acc[...] * isq, w0_ref[...],
                        preferred_element_type=jnp.float32) \
            + jnp.dot(tok_ref[...], w0_ref[...],
                      preferred_element_type=jnp.float32) * inv \
            + b0_ref[...]
        v_tok = jnp.maximum(u_tok * (gam_ref[...] * _BN_C) + bet_ref[...], 0.0)
        out_ref[...] = jnp.dot(cs_acc[...] * isq, w1_ref[...],
                               preferred_element_type=jnp.float32) \
            + jnp.dot(v_tok, w1_ref[...],
                      preferred_element_type=jnp.float32) * inv \
            + b1_ref[...]


def _stage3(parts, g, dinv, bprev, w0, b0, gam, bet, w1, b1, tok):
    full = lambda i: (0, 0)
    return pl.pallas_call(
        _stage3_tc,
        grid=(_GRID,),
        in_specs=[
            pl.BlockSpec((_NC, _BLK, _H), lambda i: (0, i, 0)),
            pl.BlockSpec((_NC, _BLK, _H), lambda i: (0, i, 0)),
            pl.BlockSpec((_BLK, 1), lambda i: (i, 0)),
            pl.BlockSpec((1, _D), full),
            pl.BlockSpec((_D, _D), full),
            pl.BlockSpec((1, _D), full),
            pl.BlockSpec((1, _D), full),
            pl.BlockSpec((1, _D), full),
            pl.BlockSpec((_D, _D), full),
            pl.BlockSpec((1, _D), full),
            pl.BlockSpec((_K, _D), full),
        ],
        out_specs=pl.BlockSpec((_K, _D), full),
        out_shape=jax.ShapeDtypeStruct((_K, _D), jnp.float32),
        scratch_shapes=[
            pltpu.VMEM((1, _D), jnp.float32),
            pltpu.VMEM((1, _D), jnp.float32),
        ],
    )(parts, g, dinv, bprev.reshape(1, _D), w0, b0.reshape(1, _D),
      gam.reshape(1, _D), bet.reshape(1, _D), w1, b1.reshape(1, _D), tok)


def kernel(x, adj_t, edge_attr, W_q, b_q, bn_g_q, bn_b_q, W_n, b_n, bn_g_n,
           bn_b_n, W_t, b_t, bn_g_t, bn_b_t, graph_token):
    del edge_attr
    _deg_sc, _prop_sc = _make_sc_kernels()

    npad = _EPAD - _E
    src2d = jnp.concatenate(
        [adj_t[0], jnp.zeros((npad,), jnp.int32)]).reshape(_EPAD // _CH, _CH)
    dst2d = jnp.concatenate(
        [adj_t[1], jnp.full((npad,), _NP - 1, jnp.int32)]).reshape(
            _EPAD // _CH, _CH)

    degp = _deg_sc(dst2d)
    dinv, g = _prelude(degp, x, W_q[0])

    parts = _prop_sc(g, src2d, dst2d)
    g = _step(parts, g, dinv, b_q[0], bn_g_q[0], bn_b_q[0], W_q[1], True)
    parts = _prop_sc(g, src2d, dst2d)
    g = _step(parts, g, dinv, b_q[1], b_q[1], b_q[1], W_n[0], False)
    parts = _prop_sc(g, src2d, dst2d)
    g = _step(parts, g, dinv, b_n[0], bn_g_n[0], bn_b_n[0], W_n[1], True)
    parts = _prop_sc(g, src2d, dst2d)
    g = _step(parts, g, dinv, b_n[1], bn_g_n[1], bn_b_n[1], W_n[2], True)
    parts = _prop_sc(g, src2d, dst2d)

    return _stage3(parts, g, dinv, b_n[2], W_t[0], b_t[0], bn_g_t[0],
                   bn_b_t[0], W_t[1], b_t[1], graph_token)
